# baseline (speedup 1.0000x reference)
import jax
import jax.numpy as jnp
import numpy as np
from jax.experimental import pallas as pl
from jax.experimental.pallas import tpu as pltpu

_LEVELS = [8, 8, 8, 5, 5, 5]
_NQ = 8
_EPS = 1e-3


def _fsq_tc_body(x_ref, win_ref, bin_ref, m_ref, bmat_ref, c_ref,
                 out_ref, idxT_ref, rnd_ref):
    hl = c_ref[:, 0:1]
    off = c_ref[:, 1:2]
    shift = c_ref[:, 2:3]

    z = jax.lax.dot_general(
        x_ref[...], win_ref[...], (((1,), (0,)), ((), ())),
        preferred_element_type=jnp.float32)
    zT = z.T + bin_ref[...]

    r = jnp.tanh(zT + shift) * hl - off
    ones_row = jnp.ones_like(zT[0:1, :])
    rnd_ref[64:65, :] = ones_row
    rnd_ref[65:72, :] = jnp.zeros_like(rnd_ref[65:72, :])
    for i in range(_NQ):
        inv = c_ref[:, 5 + i:6 + i]
        qs = c_ref[:, 13 + i:14 + i]
        zb = jnp.tanh(r * inv + shift) * hl - off
        rnd = jnp.round(zb)
        rnd_ref[i * 8:(i + 1) * 8, :] = rnd
        r = r - rnd * qs

    rnd_all = rnd_ref[...]
    idxf = jax.lax.dot_general(
        bmat_ref[...], rnd_all, (((1,), (0,)), ((), ())),
        preferred_element_type=jnp.float32)
    idxT_ref[...] = idxf.astype(jnp.int32)
    out_ref[...] = jax.lax.dot_general(
        rnd_all, m_ref[...], (((0,), (0,)), ((), ())),
        preferred_element_type=jnp.float32)


def kernel(x, W_in, b_in, W_out, b_out):
    B, N, D = x.shape
    T = B * N
    x2 = x.reshape(T, D)
    win8 = jnp.zeros((D, 8), jnp.float32).at[:, :6].set(W_in)
    bin8 = jnp.zeros((8, 1), jnp.float32).at[:6, 0].set(b_in)

    lev = jnp.array(_LEVELS + [2, 2], dtype=jnp.float32)
    half_l = (lev - 1.0) * (1.0 + _EPS) / 2.0
    offset = jnp.where(jnp.mod(lev, 2.0) == 0.0, 0.5, 0.0)
    shift = jnp.arctanh(offset / half_l)
    hw = jnp.floor(lev / 2.0)
    basis = jnp.concatenate([
        jnp.array(np.concatenate(([1], np.cumprod(_LEVELS[:-1]))),
                  dtype=jnp.float32),
        jnp.zeros((2,), jnp.float32)])
    scales = [(lev - 1.0) ** (-float(i)) for i in range(_NQ)]
    invs = [(lev - 1.0) ** float(i) for i in range(_NQ)]
    qss = [s / hw for s in scales]

    cols = [half_l, offset, shift, hw, basis]
    cols += invs
    cols += qss
    cols += [jnp.zeros((8,), jnp.float32)] * (24 - len(cols))
    consts = jnp.stack(cols, axis=1)

    wout8 = jnp.zeros((8, D), jnp.float32).at[:6, :].set(W_out)
    m_rows = jnp.concatenate([qss[i][:, None] * wout8 for i in range(_NQ)],
                             axis=0)
    m_full = jnp.concatenate([
        m_rows, b_out[None, :], jnp.zeros((7, D), jnp.float32)], axis=0)

    bmat = np.zeros((8, 72), np.float32)
    for i in range(_NQ):
        bmat[i, i * 8:i * 8 + 8] = np.concatenate(
            ([1], np.cumprod(_LEVELS[:-1]).astype(np.float64), [0, 0]))
    k0 = float(np.sum(np.floor(np.array(_LEVELS, np.float64) / 2.0)
                      * np.concatenate(([1], np.cumprod(_LEVELS[:-1])))))
    bmat[:, 64] = k0
    bmat = jnp.asarray(bmat)

    BLK = 2048
    grid = (T // BLK,)
    out, idxT = pl.pallas_call(
        _fsq_tc_body,
        grid=grid,
        in_specs=[
            pl.BlockSpec((BLK, D), lambda i: (i, 0)),
            pl.BlockSpec((D, 8), lambda i: (0, 0)),
            pl.BlockSpec((8, 1), lambda i: (0, 0)),
            pl.BlockSpec((72, D), lambda i: (0, 0)),
            pl.BlockSpec((8, 72), lambda i: (0, 0)),
            pl.BlockSpec((8, 24), lambda i: (0, 0)),
        ],
        out_specs=[
            pl.BlockSpec((BLK, D), lambda i: (i, 0)),
            pl.BlockSpec((8, BLK), lambda i: (0, i)),
        ],
        out_shape=[
            jax.ShapeDtypeStruct((T, D), jnp.float32),
            jax.ShapeDtypeStruct((8, T), jnp.int32),
        ],
        scratch_shapes=[pltpu.VMEM((72, BLK), jnp.float32)],
    )(x2, win8, bin8, m_full, bmat, consts)

    indices = idxT.T.reshape(B, N, _NQ)
    return out.reshape(B, N, D), indices
